# baseline (speedup 1.0000x reference)
import functools

import jax
import jax.numpy as jnp
import numpy as np
from jax import lax
from jax.experimental import pallas as pl
from jax.experimental.pallas import tpu as pltpu
from jax.experimental.pallas import tpu_sc as plsc

B = 16384
L = 20
E = 32
NC = 2
NS = 16
NW = NC * NS
BPW = B // NW
CH = 32
NCH = BPW // CH
ROWS = CH * L
IG = 128
TSUB = BPW // IG
KSUB = ROWS // IG

_PERM = np.stack([np.arange(16), np.arange(16) + 16], axis=1).reshape(-1)

_cached = {}


def _split(xi):
    lo = lax.bitcast_convert_type(jnp.left_shift(xi, 16), jnp.float32)
    hi = lax.bitcast_convert_type(jnp.bitwise_and(xi, jnp.int32(-65536)),
                                  jnp.float32)
    return lo, hi


def _lane_shuffle(v, perm):
    dnums = lax.GatherDimensionNumbers(
        offset_dims=(), collapsed_slice_dims=(0,), start_index_map=(0,))
    return lax.gather(v, perm[:, None], dnums, (1,),
                      mode=lax.GatherScatterMode.PROMISE_IN_BOUNDS)


def _build():
    if "k" in _cached:
        return _cached["k"]

    mesh = plsc.VectorSubcoreMesh(core_axis_name="c", subcore_axis_name="s")

    @functools.partial(
        pl.kernel,
        mesh=mesh,
        out_type=jax.ShapeDtypeStruct((B, 2 * E), jnp.float32),
        compiler_params=pltpu.CompilerParams(use_tc_tiling_on_sc=False),
        scratch_types=[
            pltpu.VMEM((BPW,), jnp.int32),
            pltpu.VMEM((ROWS + 32,), jnp.int32),
            pltpu.VMEM((ROWS, 16), jnp.int32),
            pltpu.VMEM((BPW, 16), jnp.int32),
            pltpu.VMEM((BPW, 2 * E), jnp.float32),
            pltpu.VMEM((1, 16), jnp.int32),
            pltpu.SemaphoreType.DMA,
            pltpu.SemaphoreType.DMA,
        ],
    )
    def movie_sc(title_ids, tok_flat, title_tab, token_tab, out,
                 tidx_v, cflat_v, rows_v, e1_v, out_v, row0_v, sem_t, sem_r):
        wid = lax.axis_index("s") * NC + lax.axis_index("c")
        base = wid * BPW

        pltpu.sync_copy(title_ids.at[pl.ds(base, BPW)], tidx_v)
        t_copies = [
            pltpu.async_copy(title_tab.at[tidx_v.at[pl.ds(k * IG, IG)]],
                             e1_v.at[pl.ds(k * IG, IG)], sem_t)
            for k in range(TSUB)
        ]

        pltpu.sync_copy(token_tab.at[pl.ds(0, 1)], row0_v)
        lanes = lax.iota(jnp.int32, 16)
        tailm = lanes < (L - 16)
        perms = [jnp.bitwise_xor(lanes, sh) for sh in (1, 2, 4, 8)]

        def chunk_body(c, carry):
            pltpu.sync_copy(tok_flat.at[pl.ds(base * L + c * ROWS, ROWS)],
                            cflat_v.at[pl.ds(0, ROWS)])
            r_copies = [
                pltpu.async_copy(
                    token_tab.at[cflat_v.at[pl.ds(k * IG, IG)]],
                    rows_v.at[pl.ds(k * IG, IG)], sem_r)
                for k in range(KSUB)
            ]
            for cp in r_copies:
                cp.wait()
            r0a, r0b = _split(row0_v[0, pl.ds(0, 16)])

            def title_body(b, carry2):
                r = b * L
                s0, s1 = _split(rows_v[r, pl.ds(0, 16)])
                for l in range(1, L):
                    a, bb = _split(rows_v[r + l, pl.ds(0, 16)])
                    s0 = s0 + a
                    s1 = s1 + bb
                one_v = jnp.full((16,), 1.0, jnp.float32)
                zero_v = jnp.full((16,), 0.0, jnp.float32)
                m0 = cflat_v[pl.ds(r, 16)] != 0
                m1 = (cflat_v[pl.ds(r + 16, 16)] != 0) & tailm
                nf = (jnp.where(m0, one_v, zero_v)
                      + jnp.where(m1, one_v, zero_v))
                for p in perms:
                    nf = nf + _lane_shuffle(nf, p)
                pad = jnp.float32(L) - nf
                den = jnp.maximum(nf, 1.0)
                row = c * CH + b
                out_v[row, pl.ds(E, 16)] = (s0 - pad * r0a) / den
                out_v[row, pl.ds(E + 16, 16)] = (s1 - pad * r0b) / den
                return carry2

            return lax.fori_loop(0, CH, title_body, carry)

        lax.fori_loop(0, NCH, chunk_body, 0)

        for cp in t_copies:
            cp.wait()

        def merge_body(b, carry):
            a, bb = _split(e1_v[b, pl.ds(0, 16)])
            out_v[b, pl.ds(0, 16)] = a
            out_v[b, pl.ds(16, 16)] = bb
            return carry

        lax.fori_loop(0, BPW, merge_body, 0)
        pltpu.sync_copy(out_v, out.at[pl.ds(base, BPW)])

    _cached["k"] = movie_sc
    return movie_sc


def kernel(title_ids, token_ids, title_table, token_table):
    title1d = title_ids.astype(jnp.int32).reshape(B)
    tok_flat = token_ids.astype(jnp.int32).reshape(B * L)
    tt16 = jax.lax.bitcast_convert_type(
        title_table.astype(jnp.bfloat16)[:, _PERM].reshape(-1, 16, 2),
        jnp.int32)
    kt16 = jax.lax.bitcast_convert_type(
        token_table.astype(jnp.bfloat16)[:, _PERM].reshape(-1, 16, 2),
        jnp.int32)
    return _build()(title1d, tok_flat, tt16, kt16)


# speedup vs baseline: 1.2541x; 1.2541x over previous
import functools

import jax
import jax.numpy as jnp
import numpy as np
from jax import lax
from jax.experimental import pallas as pl
from jax.experimental.pallas import tpu as pltpu
from jax.experimental.pallas import tpu_sc as plsc

B = 16384
L = 20
E = 32
NC = 2
NS = 16
NW = NC * NS
BPW = B // NW
CH = 32
NCH = BPW // CH
ROWS = CH * L
IG = 128
TSUB = BPW // IG
KSUB = ROWS // IG

_cached = {}


def _split(xi):
    lo = lax.bitcast_convert_type(jnp.left_shift(xi, 16), jnp.float32)
    hi = lax.bitcast_convert_type(jnp.bitwise_and(xi, jnp.int32(-65536)),
                                  jnp.float32)
    return lo, hi


def _lane_shuffle(v, perm):
    dnums = lax.GatherDimensionNumbers(
        offset_dims=(), collapsed_slice_dims=(0,), start_index_map=(0,))
    return lax.gather(v, perm[:, None], dnums, (1,),
                      mode=lax.GatherScatterMode.PROMISE_IN_BOUNDS)


def _unshuffle(lo, hi, half0, half1, evenm):
    h0 = jnp.where(evenm, _lane_shuffle(lo, half0), _lane_shuffle(hi, half0))
    h1 = jnp.where(evenm, _lane_shuffle(lo, half1), _lane_shuffle(hi, half1))
    return h0, h1


def _build():
    if "k" in _cached:
        return _cached["k"]

    mesh = plsc.VectorSubcoreMesh(core_axis_name="c", subcore_axis_name="s")

    @functools.partial(
        pl.kernel,
        mesh=mesh,
        out_type=jax.ShapeDtypeStruct((B, 2 * E), jnp.float32),
        compiler_params=pltpu.CompilerParams(use_tc_tiling_on_sc=False),
        scratch_types=[
            pltpu.VMEM((BPW,), jnp.int32),
            pltpu.VMEM((ROWS + 32,), jnp.int32),
            pltpu.VMEM((ROWS, 16), jnp.int32),
            pltpu.VMEM((BPW, 16), jnp.int32),
            pltpu.VMEM((BPW, 2 * E), jnp.float32),
            pltpu.VMEM((1, 16), jnp.int32),
            pltpu.SemaphoreType.DMA,
            pltpu.SemaphoreType.DMA,
        ],
    )
    def movie_sc(title_ids, tok_flat, title_tab, token_tab, out,
                 tidx_v, cflat_v, rows_v, e1_v, out_v, row0_v, sem_t, sem_r):
        wid = lax.axis_index("s") * NC + lax.axis_index("c")
        base = wid * BPW

        pltpu.sync_copy(title_ids.at[pl.ds(base, BPW)], tidx_v)
        t_copies = [
            pltpu.async_copy(title_tab.at[tidx_v.at[pl.ds(k * IG, IG)]],
                             e1_v.at[pl.ds(k * IG, IG)], sem_t)
            for k in range(TSUB)
        ]

        pltpu.sync_copy(token_tab.at[pl.ds(0, 1)], row0_v)
        lanes = lax.iota(jnp.int32, 16)
        tailm = lanes < (L - 16)
        perms = [jnp.bitwise_xor(lanes, sh) for sh in (1, 2, 4, 8)]
        half0 = lax.shift_right_logical(lanes, 1)
        half1 = half0 + 8
        evenm = jnp.bitwise_and(lanes, 1) == 0

        def chunk_body(c, carry):
            pltpu.sync_copy(tok_flat.at[pl.ds(base * L + c * ROWS, ROWS)],
                            cflat_v.at[pl.ds(0, ROWS)])
            r_copies = [
                pltpu.async_copy(
                    token_tab.at[cflat_v.at[pl.ds(k * IG, IG)]],
                    rows_v.at[pl.ds(k * IG, IG)], sem_r)
                for k in range(KSUB)
            ]
            for cp in r_copies:
                cp.wait()
            r0a, r0b = _split(row0_v[0, pl.ds(0, 16)])

            def title_body(b, carry2):
                r = b * L
                s0, s1 = _split(rows_v[r, pl.ds(0, 16)])
                for l in range(1, L):
                    a, bb = _split(rows_v[r + l, pl.ds(0, 16)])
                    s0 = s0 + a
                    s1 = s1 + bb
                one_v = jnp.full((16,), 1.0, jnp.float32)
                zero_v = jnp.full((16,), 0.0, jnp.float32)
                m0 = cflat_v[pl.ds(r, 16)] != 0
                m1 = (cflat_v[pl.ds(r + 16, 16)] != 0) & tailm
                nf = (jnp.where(m0, one_v, zero_v)
                      + jnp.where(m1, one_v, zero_v))
                for p in perms:
                    nf = nf + _lane_shuffle(nf, p)
                pad = jnp.float32(L) - nf
                den = jnp.maximum(nf, 1.0)
                elo = (s0 - pad * r0a) / den
                ehi = (s1 - pad * r0b) / den
                h0, h1 = _unshuffle(elo, ehi, half0, half1, evenm)
                row = c * CH + b
                out_v[row, pl.ds(E, 16)] = h0
                out_v[row, pl.ds(E + 16, 16)] = h1
                return carry2

            return lax.fori_loop(0, CH, title_body, carry)

        lax.fori_loop(0, NCH, chunk_body, 0)

        for cp in t_copies:
            cp.wait()

        def merge_body(b, carry):
            a, bb = _split(e1_v[b, pl.ds(0, 16)])
            h0, h1 = _unshuffle(a, bb, half0, half1, evenm)
            out_v[b, pl.ds(0, 16)] = h0
            out_v[b, pl.ds(16, 16)] = h1
            return carry

        lax.fori_loop(0, BPW, merge_body, 0)
        pltpu.sync_copy(out_v, out.at[pl.ds(base, BPW)])

    _cached["k"] = movie_sc
    return movie_sc


def kernel(title_ids, token_ids, title_table, token_table):
    title1d = title_ids.astype(jnp.int32).reshape(B)
    tok_flat = token_ids.astype(jnp.int32).reshape(B * L)
    tt16 = jax.lax.bitcast_convert_type(
        title_table.astype(jnp.bfloat16).reshape(-1, 16, 2), jnp.int32)
    kt16 = jax.lax.bitcast_convert_type(
        token_table.astype(jnp.bfloat16).reshape(-1, 16, 2), jnp.int32)
    return _build()(title1d, tok_flat, tt16, kt16)


# speedup vs baseline: 1.6355x; 1.3042x over previous
import functools

import jax
import jax.numpy as jnp
from jax import lax
from jax.experimental import pallas as pl
from jax.experimental.pallas import tpu as pltpu
from jax.experimental.pallas import tpu_sc as plsc

B = 16384
L = 20
E = 32
NC = 2
NS = 16
NW = NC * NS
BPW = B // NW
CH = 32
NCH = BPW // CH
ROWS = CH * L
IG = 128
KSUB = ROWS // IG
TCH = 128
NTCH = BPW // TCH
VT = 1000004 // 4

_cached = {}


def _lane_shuffle(v, perm):
    dnums = lax.GatherDimensionNumbers(
        offset_dims=(), collapsed_slice_dims=(0,), start_index_map=(0,))
    return lax.gather(v, perm[:, None], dnums, (1,),
                      mode=lax.GatherScatterMode.PROMISE_IN_BOUNDS)


def _build_title():
    if "kt" in _cached:
        return _cached["kt"]

    mesh = plsc.VectorSubcoreMesh(core_axis_name="c", subcore_axis_name="s")

    @functools.partial(
        pl.kernel,
        mesh=mesh,
        out_type=jax.ShapeDtypeStruct((B, IG), jnp.float32),
        compiler_params=pltpu.CompilerParams(use_tc_tiling_on_sc=True),
        scratch_types=[
            pltpu.VMEM((BPW,), jnp.int32),
            pltpu.VMEM((TCH,), jnp.int32),
            pltpu.VMEM((TCH, IG), jnp.float32),
            pltpu.VMEM((TCH, IG), jnp.float32),
            pltpu.SemaphoreType.DMA,
        ],
    )
    def title_sc(title_ids, tab128, out, tidx_v, gidx_v, trow_v, outc_v, sem):
        wid = lax.axis_index("s") * NC + lax.axis_index("c")
        base = wid * BPW
        pltpu.sync_copy(title_ids.at[pl.ds(base, BPW)], tidx_v)
        lanes = lax.iota(jnp.int32, 16)

        def chunk_body(c, carry):
            def prep(g, carry2):
                ids = tidx_v[pl.ds(c * TCH + g * 16, 16)]
                gidx_v[pl.ds(g * 16, 16)] = lax.shift_right_logical(ids, 2)
                return carry2

            lax.fori_loop(0, TCH // 16, prep, 0)
            pltpu.async_copy(tab128.at[gidx_v], trow_v, sem).wait()

            def extract(g, carry2):
                ids = tidx_v[pl.ds(c * TCH + g * 16, 16)]
                for j in range(16):
                    colb = jnp.bitwise_and(ids[j], 3) * E
                    rowp = g * 16 + j
                    outc_v[rowp, pl.ds(0, 16)] = trow_v[rowp, pl.ds(colb, 16)]
                    outc_v[rowp, pl.ds(16, 16)] = trow_v[rowp,
                                                         pl.ds(colb + 16, 16)]
                return carry2

            lax.fori_loop(0, TCH // 16, extract, 0)
            pltpu.sync_copy(outc_v, out.at[pl.ds(base + c * TCH, TCH)])
            return carry

        lax.fori_loop(0, NTCH, chunk_body, 0)

    _cached["kt"] = title_sc
    return title_sc


def _build_tok():
    if "kk" in _cached:
        return _cached["kk"]

    mesh = plsc.VectorSubcoreMesh(core_axis_name="c", subcore_axis_name="s")

    @functools.partial(
        pl.kernel,
        mesh=mesh,
        out_type=jax.ShapeDtypeStruct((B, E), jnp.float32),
        compiler_params=pltpu.CompilerParams(use_tc_tiling_on_sc=False),
        scratch_types=[
            pltpu.VMEM((ROWS + 32,), jnp.int32),
            pltpu.VMEM((ROWS, E), jnp.float32),
            pltpu.VMEM((BPW, E), jnp.float32),
            pltpu.VMEM((1, E), jnp.float32),
            pltpu.SemaphoreType.DMA,
        ],
    )
    def tok_sc(tok_flat, token_tab, out, cflat_v, rows_v, out_v, row0_v, sem):
        wid = lax.axis_index("s") * NC + lax.axis_index("c")
        base = wid * BPW
        pltpu.sync_copy(token_tab.at[pl.ds(0, 1)], row0_v)
        lanes = lax.iota(jnp.int32, 16)
        tailm = lanes < (L - 16)
        perms = [jnp.bitwise_xor(lanes, sh) for sh in (1, 2, 4, 8)]

        def chunk_body(c, carry):
            pltpu.sync_copy(tok_flat.at[pl.ds(base * L + c * ROWS, ROWS)],
                            cflat_v.at[pl.ds(0, ROWS)])
            r_copies = [
                pltpu.async_copy(
                    token_tab.at[cflat_v.at[pl.ds(k * IG, IG)]],
                    rows_v.at[pl.ds(k * IG, IG)], sem)
                for k in range(KSUB)
            ]
            for cp in r_copies:
                cp.wait()
            r0a = row0_v[0, pl.ds(0, 16)]
            r0b = row0_v[0, pl.ds(16, 16)]

            def title_body(b, carry2):
                r = b * L
                s0 = rows_v[r, pl.ds(0, 16)]
                s1 = rows_v[r, pl.ds(16, 16)]
                for l in range(1, L):
                    s0 = s0 + rows_v[r + l, pl.ds(0, 16)]
                    s1 = s1 + rows_v[r + l, pl.ds(16, 16)]
                one_v = jnp.full((16,), 1.0, jnp.float32)
                zero_v = jnp.full((16,), 0.0, jnp.float32)
                m0 = cflat_v[pl.ds(r, 16)] != 0
                m1 = (cflat_v[pl.ds(r + 16, 16)] != 0) & tailm
                nf = (jnp.where(m0, one_v, zero_v)
                      + jnp.where(m1, one_v, zero_v))
                for p in perms:
                    nf = nf + _lane_shuffle(nf, p)
                pad = jnp.float32(L) - nf
                den = jnp.maximum(nf, 1.0)
                row = c * CH + b
                out_v[row, pl.ds(0, 16)] = (s0 - pad * r0a) / den
                out_v[row, pl.ds(16, 16)] = (s1 - pad * r0b) / den
                return carry2

            return lax.fori_loop(0, CH, title_body, carry)

        lax.fori_loop(0, NCH, chunk_body, 0)
        pltpu.sync_copy(out_v, out.at[pl.ds(base, BPW)])

    _cached["kk"] = tok_sc
    return tok_sc


def kernel(title_ids, token_ids, title_table, token_table):
    title1d = title_ids.astype(jnp.int32).reshape(B)
    tok_flat = token_ids.astype(jnp.int32).reshape(B * L)
    tab128 = jnp.pad(title_table, ((0, 3), (0, 0))).reshape(VT, IG)
    e1 = _build_title()(title1d, tab128)[:, :E]
    e2 = _build_tok()(tok_flat, token_table)
    return jnp.concatenate([e1, e2], axis=1)


# speedup vs baseline: 3.5951x; 2.1981x over previous
import functools

import jax
import jax.numpy as jnp
from jax import lax
from jax.experimental import pallas as pl
from jax.experimental.pallas import tpu as pltpu
from jax.experimental.pallas import tpu_sc as plsc

B = 16384
L = 20
E = 32
NC = 2
NS = 16
NW = NC * NS
BPW = B // NW
CH = 32
NCH = BPW // CH
ROWS = CH * L
IG = 128
KSUB = ROWS // IG
TCH = 128
NTCH = BPW // TCH
VT = 1000004 // 4

_cached = {}


def _lane_shuffle(v, perm):
    dnums = lax.GatherDimensionNumbers(
        offset_dims=(), collapsed_slice_dims=(0,), start_index_map=(0,))
    return lax.gather(v, perm[:, None], dnums, (1,),
                      mode=lax.GatherScatterMode.PROMISE_IN_BOUNDS)


def _build_title():
    if "kt" in _cached:
        return _cached["kt"]

    mesh = plsc.VectorSubcoreMesh(core_axis_name="c", subcore_axis_name="s")

    @functools.partial(
        pl.kernel,
        mesh=mesh,
        out_type=jax.ShapeDtypeStruct((B, IG), jnp.float32),
        compiler_params=pltpu.CompilerParams(use_tc_tiling_on_sc=True),
        scratch_types=[
            pltpu.VMEM((BPW,), jnp.int32),
            pltpu.VMEM((TCH, E), jnp.float32),
            pltpu.VMEM((TCH, IG), jnp.float32),
            pltpu.SemaphoreType.DMA,
        ],
    )
    def title_sc(title_ids, tab, out, tidx_v, slots_v, outc_v, sem):
        wid = lax.axis_index("s") * NC + lax.axis_index("c")
        base = wid * BPW
        pltpu.sync_copy(title_ids.at[pl.ds(base, BPW)], tidx_v)

        def chunk_body(c, carry):
            def group_body(g, carry2):
                ids = tidx_v[pl.ds(c * TCH + g * 16, 16)]
                copies = []
                idjs = []
                for j in range(16):
                    idj = ids[j]
                    idjs.append(idj)
                    t8 = pl.multiple_of(
                        jnp.bitwise_and(idj, jnp.int32(~7)), 8)
                    copies.append(pltpu.async_copy(
                        tab.at[pl.ds(t8, 8), :],
                        slots_v.at[pl.ds(j * 8, 8), :], sem))
                for cp in copies:
                    cp.wait()
                for j in range(16):
                    srow = j * 8 + jnp.bitwise_and(idjs[j], 7)
                    rowp = g * 16 + j
                    outc_v[rowp, pl.ds(0, 16)] = slots_v[srow, pl.ds(0, 16)]
                    outc_v[rowp, pl.ds(16, 16)] = slots_v[srow, pl.ds(16, 16)]
                return carry2

            lax.fori_loop(0, TCH // 16, group_body, 0)
            pltpu.sync_copy(outc_v, out.at[pl.ds(base + c * TCH, TCH)])
            return carry

        lax.fori_loop(0, NTCH, chunk_body, 0)

    _cached["kt"] = title_sc
    return title_sc


def _build_tok():
    if "kk" in _cached:
        return _cached["kk"]

    mesh = plsc.VectorSubcoreMesh(core_axis_name="c", subcore_axis_name="s")

    @functools.partial(
        pl.kernel,
        mesh=mesh,
        out_type=jax.ShapeDtypeStruct((B, E), jnp.float32),
        compiler_params=pltpu.CompilerParams(use_tc_tiling_on_sc=False),
        scratch_types=[
            pltpu.VMEM((ROWS + 32,), jnp.int32),
            pltpu.VMEM((ROWS, E), jnp.float32),
            pltpu.VMEM((BPW, E), jnp.float32),
            pltpu.VMEM((1, E), jnp.float32),
            pltpu.SemaphoreType.DMA,
        ],
    )
    def tok_sc(tok_flat, token_tab, out, cflat_v, rows_v, out_v, row0_v, sem):
        wid = lax.axis_index("s") * NC + lax.axis_index("c")
        base = wid * BPW
        pltpu.sync_copy(token_tab.at[pl.ds(0, 1)], row0_v)
        lanes = lax.iota(jnp.int32, 16)
        tailm = lanes < (L - 16)
        perms = [jnp.bitwise_xor(lanes, sh) for sh in (1, 2, 4, 8)]

        def chunk_body(c, carry):
            pltpu.sync_copy(tok_flat.at[pl.ds(base * L + c * ROWS, ROWS)],
                            cflat_v.at[pl.ds(0, ROWS)])
            r_copies = [
                pltpu.async_copy(
                    token_tab.at[cflat_v.at[pl.ds(k * IG, IG)]],
                    rows_v.at[pl.ds(k * IG, IG)], sem)
                for k in range(KSUB)
            ]
            for cp in r_copies:
                cp.wait()
            r0a = row0_v[0, pl.ds(0, 16)]
            r0b = row0_v[0, pl.ds(16, 16)]

            def title_body(b, carry2):
                r = b * L
                s0 = rows_v[r, pl.ds(0, 16)]
                s1 = rows_v[r, pl.ds(16, 16)]
                for l in range(1, L):
                    s0 = s0 + rows_v[r + l, pl.ds(0, 16)]
                    s1 = s1 + rows_v[r + l, pl.ds(16, 16)]
                one_v = jnp.full((16,), 1.0, jnp.float32)
                zero_v = jnp.full((16,), 0.0, jnp.float32)
                m0 = cflat_v[pl.ds(r, 16)] != 0
                m1 = (cflat_v[pl.ds(r + 16, 16)] != 0) & tailm
                nf = (jnp.where(m0, one_v, zero_v)
                      + jnp.where(m1, one_v, zero_v))
                for p in perms:
                    nf = nf + _lane_shuffle(nf, p)
                pad = jnp.float32(L) - nf
                den = jnp.maximum(nf, 1.0)
                row = c * CH + b
                out_v[row, pl.ds(0, 16)] = (s0 - pad * r0a) / den
                out_v[row, pl.ds(16, 16)] = (s1 - pad * r0b) / den
                return carry2

            return lax.fori_loop(0, CH, title_body, carry)

        lax.fori_loop(0, NCH, chunk_body, 0)
        pltpu.sync_copy(out_v, out.at[pl.ds(base, BPW)])

    _cached["kk"] = tok_sc
    return tok_sc


def kernel(title_ids, token_ids, title_table, token_table):
    title1d = title_ids.astype(jnp.int32).reshape(B)
    tok_flat = token_ids.astype(jnp.int32).reshape(B * L)
    e1 = _build_title()(title1d, title_table)[:, :E]
    e2 = _build_tok()(tok_flat, token_table)
    return jnp.concatenate([e1, e2], axis=1)


# speedup vs baseline: 3.5962x; 1.0003x over previous
import functools

import jax
import jax.numpy as jnp
from jax import lax
from jax.experimental import pallas as pl
from jax.experimental.pallas import tpu as pltpu
from jax.experimental.pallas import tpu_sc as plsc

B = 16384
L = 20
E = 32
NC = 2
NS = 16
NW = NC * NS
BPW = B // NW
CH = 32
NCH = BPW // CH
ROWS = CH * L
IG = 128
KSUB = ROWS // IG
TCH = 128
NTCH = BPW // TCH

_cached = {}


def _lane_shuffle(v, perm):
    dnums = lax.GatherDimensionNumbers(
        offset_dims=(), collapsed_slice_dims=(0,), start_index_map=(0,))
    return lax.gather(v, perm[:, None], dnums, (1,),
                      mode=lax.GatherScatterMode.PROMISE_IN_BOUNDS)


def _build_title():
    if "kt" in _cached:
        return _cached["kt"]

    mesh = plsc.VectorSubcoreMesh(core_axis_name="c", subcore_axis_name="s")

    @functools.partial(
        pl.kernel,
        mesh=mesh,
        out_type=jax.ShapeDtypeStruct((B, IG), jnp.float32),
        compiler_params=pltpu.CompilerParams(use_tc_tiling_on_sc=True),
        scratch_types=[
            pltpu.VMEM((BPW,), jnp.int32),
            pltpu.VMEM((TCH, E), jnp.float32),
            pltpu.VMEM((TCH, IG), jnp.float32),
            pltpu.SemaphoreType.DMA,
        ],
    )
    def title_sc(title_ids, tab, out, tidx_v, slots_v, outc_v, sem):
        wid = lax.axis_index("s") * NC + lax.axis_index("c")
        base = wid * BPW
        pltpu.sync_copy(title_ids.at[pl.ds(base, BPW)], tidx_v)

        def chunk_body(c, carry):
            def group_body(g, carry2):
                ids = tidx_v[pl.ds(c * TCH + g * 16, 16)]
                copies = []
                idjs = []
                for j in range(16):
                    idj = ids[j]
                    idjs.append(idj)
                    t8 = pl.multiple_of(
                        jnp.bitwise_and(idj, jnp.int32(~7)), 8)
                    copies.append(pltpu.async_copy(
                        tab.at[pl.ds(t8, 8), :],
                        slots_v.at[pl.ds(j * 8, 8), :], sem))
                for cp in copies:
                    cp.wait()
                for j in range(16):
                    srow = j * 8 + jnp.bitwise_and(idjs[j], 7)
                    rowp = g * 16 + j
                    outc_v[rowp, pl.ds(0, 16)] = slots_v[srow, pl.ds(0, 16)]
                    outc_v[rowp, pl.ds(16, 16)] = slots_v[srow, pl.ds(16, 16)]
                return carry2

            lax.fori_loop(0, TCH // 16, group_body, 0)
            pltpu.sync_copy(outc_v, out.at[pl.ds(base + c * TCH, TCH)])
            return carry

        lax.fori_loop(0, NTCH, chunk_body, 0)

    _cached["kt"] = title_sc
    return title_sc


def _build_tok():
    if "kk" in _cached:
        return _cached["kk"]

    mesh = plsc.VectorSubcoreMesh(core_axis_name="c", subcore_axis_name="s")

    @functools.partial(
        pl.kernel,
        mesh=mesh,
        out_type=jax.ShapeDtypeStruct((B, E), jnp.float32),
        compiler_params=pltpu.CompilerParams(use_tc_tiling_on_sc=False),
        scratch_types=[
            pltpu.VMEM((ROWS + 32,), jnp.int32),
            pltpu.VMEM((ROWS, E), jnp.float32),
            pltpu.VMEM((BPW, E), jnp.float32),
            pltpu.VMEM((1, E), jnp.float32),
            pltpu.SemaphoreType.DMA,
        ],
    )
    def tok_sc(tok_flat, token_tab, out, cflat_v, rows_v, out_v, row0_v, sem):
        wid = lax.axis_index("s") * NC + lax.axis_index("c")
        base = wid * BPW
        pltpu.sync_copy(token_tab.at[pl.ds(0, 1)], row0_v)
        lanes = lax.iota(jnp.int32, 16)
        tailm = lanes < (L - 16)
        perms = [jnp.bitwise_xor(lanes, sh) for sh in (1, 2, 4, 8)]

        def chunk_body(c, carry):
            pltpu.sync_copy(tok_flat.at[pl.ds(base * L + c * ROWS, ROWS)],
                            cflat_v.at[pl.ds(0, ROWS)])
            r_copies = [
                pltpu.async_copy(
                    token_tab.at[cflat_v.at[pl.ds(k * IG, IG)]],
                    rows_v.at[pl.ds(k * IG, IG)], sem)
                for k in range(KSUB)
            ]
            for cp in r_copies:
                cp.wait()
            r0a = row0_v[0, pl.ds(0, 16)]
            r0b = row0_v[0, pl.ds(16, 16)]

            def title_body(b, carry2):
                r = b * L
                s0 = rows_v[r, pl.ds(0, 16)]
                s1 = rows_v[r, pl.ds(16, 16)]
                for l in range(1, L):
                    s0 = s0 + rows_v[r + l, pl.ds(0, 16)]
                    s1 = s1 + rows_v[r + l, pl.ds(16, 16)]
                one_v = jnp.full((16,), 1.0, jnp.float32)
                zero_v = jnp.full((16,), 0.0, jnp.float32)
                m0 = cflat_v[pl.ds(r, 16)] != 0
                m1 = (cflat_v[pl.ds(r + 16, 16)] != 0) & tailm
                nf = (jnp.where(m0, one_v, zero_v)
                      + jnp.where(m1, one_v, zero_v))
                for p in perms:
                    nf = nf + _lane_shuffle(nf, p)
                pad = jnp.float32(L) - nf
                den = jnp.maximum(nf, 1.0)
                row = c * CH + b
                out_v[row, pl.ds(0, 16)] = (s0 - pad * r0a) / den
                out_v[row, pl.ds(16, 16)] = (s1 - pad * r0b) / den
                return carry2

            return lax.fori_loop(0, CH, title_body, carry)

        lax.fori_loop(0, NCH, chunk_body, 0)
        pltpu.sync_copy(out_v, out.at[pl.ds(base, BPW)])

    _cached["kk"] = tok_sc
    return tok_sc


def kernel(title_ids, token_ids, title_table, token_table):
    title1d = title_ids.astype(jnp.int32).reshape(B)
    tok_flat = token_ids.astype(jnp.int32).reshape(B * L)
    e1 = _build_title()(title1d, title_table)[:, :E]
    e2 = _build_tok()(tok_flat, token_table)
    return jnp.concatenate([e1, e2], axis=1)
